# baseline (speedup 1.0000x reference)
import functools

import jax
import jax.numpy as jnp
from jax import lax
from jax.experimental import pallas as pl
from jax.experimental.pallas import tpu as pltpu
from jax.experimental.pallas import tpu_sc as plsc

B = 32; L = 12; N = 883; C = 3
NP = 896
TOPK = 20
TOD = 288; DOW = 7; SEQ_OUT = 12
HID = 128


_GW = 32
_GCHUNKS = NP // 128


def _sc_gather(table, idx):
    mesh = plsc.VectorSubcoreMesh(core_axis_name="c", subcore_axis_name="s")

    @functools.partial(
        pl.kernel, mesh=mesh,
        out_type=jax.ShapeDtypeStruct((B, NP, 128), jnp.float32),
        scratch_types=[
            pltpu.VMEM((_GCHUNKS, 128), jnp.int32),
            pltpu.VMEM((NP, 128), jnp.float32),
            pltpu.SemaphoreType.DMA,
        ],
    )
    def k(table_hbm, idx_hbm, out_hbm, idx_v, rows_v, sem):
        wid = lax.axis_index("s") * 2 + lax.axis_index("c")
        pltpu.sync_copy(idx_hbm.at[wid], idx_v)
        copies = [
            pltpu.async_copy(table_hbm.at[idx_v.at[j]],
                             rows_v.at[pl.ds(j * 128, 128)], sem)
            for j in range(_GCHUNKS)
        ]
        for c in copies:
            c.wait()
        pltpu.sync_copy(rows_v, out_hbm.at[wid])

    return k(table, idx)


def _kth_largest(x, k, nbits=30):
    rows = x.shape[0]
    kf = jnp.float32(k)

    def cond(carry):
        _, _, _, i, done = carry
        return jnp.logical_and(i < nbits, jnp.logical_not(done))

    def body(carry):
        lo, hi, clo, i, _ = carry
        mid = lo + (hi - lo) // 2
        t = lax.bitcast_convert_type(mid, jnp.float32)
        c = jnp.sum((x >= t).astype(jnp.float32), axis=1, keepdims=True)
        ge = c >= kf
        lo = jnp.where(ge, mid, lo)
        hi = jnp.where(ge, hi, mid)
        clo = jnp.where(ge, c, clo)
        done = jnp.all((clo == kf) | (hi - lo <= 1))
        return lo, hi, clo, i + 1, done

    lo0 = jnp.zeros((rows, 1), jnp.int32)
    hi0 = jnp.full((rows, 1), 0x3F800001, jnp.int32)
    c0 = jnp.full((rows, 1), jnp.float32(x.shape[1]))
    lo, _, _, _, _ = lax.while_loop(
        cond, body, (lo0, hi0, c0, jnp.int32(0), jnp.bool_(False)))
    return lax.bitcast_convert_type(lo, jnp.float32)


def _main_body(hist_ref, td_ref, ne_ref, neu_ref, ned_ref, e1_ref,
               Wts_ref, bts_ref,
               W1a_ref, b1a_ref, W1b_ref, b1b_ref, W1c_ref, b1c_ref,
               Wf_ref, bf_ref, out_ref, static_scr):
    b = pl.program_id(0)

    @pl.when(b == 0)
    def _():
        r = lax.dot_general(ned_ref[...], neu_ref[...],
                            (((1,), (1,)), ((), ())),
                            preferred_element_type=jnp.float32)
        col = lax.broadcasted_iota(jnp.int32, (NP, NP), 1)
        valid = col < N
        r = jnp.where(valid, jnp.maximum(r, 0.0), -1e30)
        m = jnp.max(r, axis=1, keepdims=True)
        e = jnp.exp(r - m)
        sg = e / jnp.sum(e, axis=1, keepdims=True)
        thr = _kth_largest(sg, TOPK)
        gt = sg > thr
        ties = (sg == thr) & valid
        row_i = lax.broadcasted_iota(jnp.int32, (NP, NP), 0)
        tri = (row_i <= col).astype(jnp.float32)
        rank = lax.dot_general(ties.astype(jnp.float32), tri,
                               (((1,), (0,)), ((), ())),
                               preferred_element_type=jnp.float32)
        need = jnp.float32(TOPK) - jnp.sum(gt.astype(jnp.float32), axis=1,
                                           keepdims=True)
        keep = gt | (ties & (rank <= need))
        eye = (row_i == col).astype(jnp.float32)
        static_scr[...] = jnp.where(keep, sg, 0.0) + eye

    ts = jnp.dot(hist_ref[0], Wts_ref[...],
                 preferred_element_type=jnp.float32) + bts_ref[...]
    H = jnp.concatenate([ts, ne_ref[...], td_ref[0][:, :64]],
                        axis=1)

    h1 = jnp.maximum(jnp.dot(H, W1a_ref[...],
                             preferred_element_type=jnp.float32)
                     + b1a_ref[...], 0.0)
    h2 = jnp.maximum(jnp.dot(h1, W1b_ref[...],
                             preferred_element_type=jnp.float32)
                     + b1b_ref[...], 0.0)
    f1 = jnp.dot(h2, W1c_ref[...],
                 preferred_element_type=jnp.float32) + b1c_ref[...]
    nv = jnp.tanh(e1_ref[...] * f1)

    a = lax.dot_general(nv, nv, (((1,), (1,)), ((), ())),
                        preferred_element_type=jnp.float32)
    adj = jnp.maximum(jnp.tanh(a), 0.0)
    thr = _kth_largest(adj, TOPK)
    dyn = jnp.where(adj >= thr, adj, 0.0)

    hs = jnp.dot(static_scr[...], H, preferred_element_type=jnp.float32)
    hd = jnp.dot(dyn, H, preferred_element_type=jnp.float32)
    fused = jnp.maximum(hs + hd, 0.0)
    out_ref[0] = jnp.dot(fused, Wf_ref[...],
                         preferred_element_type=jnp.float32) + bf_ref[...]


def kernel(history_data, TID, DIW, node_emb, node_emb_u, node_emb_d,
           emb1_w, emb2_w, Wts, bts, W1a, b1a, W1b, b1b, W1c, b1c,
           W2a, b2a, W2b, b2b, W2c, b2c, W_fore, b_fore):
    f32 = jnp.float32
    tid_idx = (history_data[:, -1, :, 1] * TOD).astype(jnp.int32)
    diw_idx = (history_data[:, -1, :, 2] * DOW).astype(jnp.int32)
    pad_n = NP - N
    table = jnp.concatenate([
        jnp.broadcast_to(TID[:, None, :], (TOD, DOW, 32)),
        jnp.broadcast_to(DIW[None, :, :], (TOD, DOW, 32)),
    ], axis=-1).reshape(TOD * DOW, 64)
    table = jnp.pad(table, ((0, 0), (0, 64)))
    comb_idx = jnp.pad(tid_idx * DOW + diw_idx,
                       ((0, 0), (0, pad_n))).reshape(B, NP // 128, 128)
    td = _sc_gather(table, comb_idx)
    hist2 = history_data.transpose(0, 2, 1, 3).reshape(B, N, L * C)
    hist2 = jnp.pad(hist2, ((0, 0), (0, pad_n), (0, 0)))
    ne_p = jnp.pad(node_emb, ((0, pad_n), (0, 0)))
    neu_p = jnp.pad(node_emb_u, ((0, pad_n), (0, 0)))
    ned_p = jnp.pad(node_emb_d, ((0, pad_n), (0, 0)))
    e1_p = jnp.pad(emb1_w, ((0, pad_n), (0, 0)))

    full = lambda shape: pl.BlockSpec(shape, lambda b: (0,) * len(shape))
    perb2 = lambda shape: pl.BlockSpec((1,) + shape, lambda b: (b, 0, 0))

    out = pl.pallas_call(
        _main_body,
        grid=(B,),
        in_specs=[
            perb2((NP, L * C)),
            perb2((NP, 128)),
            full((NP, 32)),
            full((NP, 32)),
            full((NP, 32)),
            full((NP, 40)),
            full((L * C, 32)),
            full((1, 32)),
            full((HID, 64)),
            full((1, 64)),
            full((64, 64)),
            full((1, 64)),
            full((64, 40)),
            full((1, 40)),
            full((HID, SEQ_OUT)),
            full((1, SEQ_OUT)),
        ],
        out_specs=perb2((NP, SEQ_OUT)),
        out_shape=jax.ShapeDtypeStruct((B, NP, SEQ_OUT), f32),
        scratch_shapes=[pltpu.VMEM((NP, NP), f32)],
    )(hist2, td, ne_p, neu_p, ned_p, e1_p,
      Wts, bts[None, :], W1a, b1a[None, :], W1b, b1b[None, :],
      W1c, b1c[None, :], W_fore, b_fore[None, :])
    return out[:, :N, :]


# speedup vs baseline: 1.1520x; 1.1520x over previous
import functools

import jax
import jax.numpy as jnp
from jax import lax
from jax.experimental import pallas as pl
from jax.experimental.pallas import tpu as pltpu
from jax.experimental.pallas import tpu_sc as plsc

B = 32; L = 12; N = 883; C = 3
NP = 896
TOPK = 20
TOD = 288; DOW = 7; SEQ_OUT = 12
HID = 128


_GW = 32
_GCHUNKS = NP // 128


def _sc_gather(table, idx):
    mesh = plsc.VectorSubcoreMesh(core_axis_name="c", subcore_axis_name="s")

    @functools.partial(
        pl.kernel, mesh=mesh,
        out_type=jax.ShapeDtypeStruct((B, NP, 128), jnp.float32),
        scratch_types=[
            pltpu.VMEM((_GCHUNKS, 128), jnp.int32),
            pltpu.VMEM((NP, 128), jnp.float32),
            pltpu.SemaphoreType.DMA,
        ],
    )
    def k(table_hbm, idx_hbm, out_hbm, idx_v, rows_v, sem):
        wid = lax.axis_index("s") * 2 + lax.axis_index("c")
        pltpu.sync_copy(idx_hbm.at[wid], idx_v)
        copies = [
            pltpu.async_copy(table_hbm.at[idx_v.at[j]],
                             rows_v.at[pl.ds(j * 128, 128)], sem)
            for j in range(_GCHUNKS)
        ]
        for c in copies:
            c.wait()
        pltpu.sync_copy(rows_v, out_hbm.at[wid])

    return k(table, idx)


def _kth_largest(x, k, nbits=30):
    rows = x.shape[0]
    kf = jnp.float32(k)

    def body(_, carry):
        lo, hi = carry
        mid = lo + (hi - lo) // 2
        t = lax.bitcast_convert_type(mid, jnp.float32)
        c = jnp.sum((x >= t).astype(jnp.float32), axis=1, keepdims=True)
        ge = c >= kf
        return jnp.where(ge, mid, lo), jnp.where(ge, hi, mid)

    lo0 = jnp.zeros((rows, 1), jnp.int32)
    hi0 = jnp.full((rows, 1), 0x3F800001, jnp.int32)
    lo, _ = lax.fori_loop(0, nbits, body, (lo0, hi0))
    return lax.bitcast_convert_type(lo, jnp.float32)


def _main_body(hist_ref, td_ref, ne_ref, neu_ref, ned_ref, e1_ref,
               Wts_ref, bts_ref,
               W1a_ref, b1a_ref, W1b_ref, b1b_ref, W1c_ref, b1c_ref,
               Wf_ref, bf_ref, out_ref, static_scr):
    b = pl.program_id(0)

    @pl.when(b == 0)
    def _():
        r = lax.dot_general(ned_ref[...], neu_ref[...],
                            (((1,), (1,)), ((), ())),
                            preferred_element_type=jnp.float32)
        col = lax.broadcasted_iota(jnp.int32, (NP, NP), 1)
        valid = col < N
        r = jnp.where(valid, jnp.maximum(r, 0.0), -1e30)
        m = jnp.max(r, axis=1, keepdims=True)
        e = jnp.exp(r - m)
        sg = e / jnp.sum(e, axis=1, keepdims=True)
        thr = _kth_largest(sg, TOPK)
        gt = sg > thr
        ties = (sg == thr) & valid
        row_i = lax.broadcasted_iota(jnp.int32, (NP, NP), 0)
        tri = (row_i <= col).astype(jnp.float32)
        rank = lax.dot_general(ties.astype(jnp.float32), tri,
                               (((1,), (0,)), ((), ())),
                               preferred_element_type=jnp.float32)
        need = jnp.float32(TOPK) - jnp.sum(gt.astype(jnp.float32), axis=1,
                                           keepdims=True)
        keep = gt | (ties & (rank <= need))
        eye = (row_i == col).astype(jnp.float32)
        static_scr[...] = jnp.where(keep, sg, 0.0) + eye

    ts = jnp.dot(hist_ref[0], Wts_ref[...],
                 preferred_element_type=jnp.float32) + bts_ref[...]
    H = jnp.concatenate([ts, ne_ref[...], td_ref[0][:, :64]],
                        axis=1)

    h1 = jnp.maximum(jnp.dot(H, W1a_ref[...],
                             preferred_element_type=jnp.float32)
                     + b1a_ref[...], 0.0)
    h2 = jnp.maximum(jnp.dot(h1, W1b_ref[...],
                             preferred_element_type=jnp.float32)
                     + b1b_ref[...], 0.0)
    f1 = jnp.dot(h2, W1c_ref[...],
                 preferred_element_type=jnp.float32) + b1c_ref[...]
    nv = jnp.tanh(e1_ref[...] * f1)

    a = lax.dot_general(nv, nv, (((1,), (1,)), ((), ())),
                        preferred_element_type=jnp.float32)
    adj = jnp.maximum(jnp.tanh(a), 0.0)
    thr = _kth_largest(adj, TOPK)
    dyn = jnp.where(adj >= thr, adj, 0.0)

    hs = jnp.dot(static_scr[...], H, preferred_element_type=jnp.float32)
    hd = jnp.dot(dyn, H, preferred_element_type=jnp.float32)
    fused = jnp.maximum(hs + hd, 0.0)
    out_ref[0] = jnp.dot(fused, Wf_ref[...],
                         preferred_element_type=jnp.float32) + bf_ref[...]


def kernel(history_data, TID, DIW, node_emb, node_emb_u, node_emb_d,
           emb1_w, emb2_w, Wts, bts, W1a, b1a, W1b, b1b, W1c, b1c,
           W2a, b2a, W2b, b2b, W2c, b2c, W_fore, b_fore):
    f32 = jnp.float32
    tid_idx = (history_data[:, -1, :, 1] * TOD).astype(jnp.int32)
    diw_idx = (history_data[:, -1, :, 2] * DOW).astype(jnp.int32)
    pad_n = NP - N
    table = jnp.concatenate([
        jnp.broadcast_to(TID[:, None, :], (TOD, DOW, 32)),
        jnp.broadcast_to(DIW[None, :, :], (TOD, DOW, 32)),
    ], axis=-1).reshape(TOD * DOW, 64)
    table = jnp.pad(table, ((0, 0), (0, 64)))
    comb_idx = jnp.pad(tid_idx * DOW + diw_idx,
                       ((0, 0), (0, pad_n))).reshape(B, NP // 128, 128)
    td = _sc_gather(table, comb_idx)
    hist2 = history_data.transpose(0, 2, 1, 3).reshape(B, N, L * C)
    hist2 = jnp.pad(hist2, ((0, 0), (0, pad_n), (0, 0)))
    ne_p = jnp.pad(node_emb, ((0, pad_n), (0, 0)))
    neu_p = jnp.pad(node_emb_u, ((0, pad_n), (0, 0)))
    ned_p = jnp.pad(node_emb_d, ((0, pad_n), (0, 0)))
    e1_p = jnp.pad(emb1_w, ((0, pad_n), (0, 0)))

    full = lambda shape: pl.BlockSpec(shape, lambda b: (0,) * len(shape))
    perb2 = lambda shape: pl.BlockSpec((1,) + shape, lambda b: (b, 0, 0))

    out = pl.pallas_call(
        _main_body,
        grid=(B,),
        in_specs=[
            perb2((NP, L * C)),
            perb2((NP, 128)),
            full((NP, 32)),
            full((NP, 32)),
            full((NP, 32)),
            full((NP, 40)),
            full((L * C, 32)),
            full((1, 32)),
            full((HID, 64)),
            full((1, 64)),
            full((64, 64)),
            full((1, 64)),
            full((64, 40)),
            full((1, 40)),
            full((HID, SEQ_OUT)),
            full((1, SEQ_OUT)),
        ],
        out_specs=perb2((NP, SEQ_OUT)),
        out_shape=jax.ShapeDtypeStruct((B, NP, SEQ_OUT), f32),
        scratch_shapes=[pltpu.VMEM((NP, NP), f32)],
    )(hist2, td, ne_p, neu_p, ned_p, e1_p,
      Wts, bts[None, :], W1a, b1a[None, :], W1b, b1b[None, :],
      W1c, b1c[None, :], W_fore, b_fore[None, :])
    return out[:, :N, :]


# speedup vs baseline: 2.0230x; 1.7560x over previous
import functools

import jax
import jax.numpy as jnp
from jax import lax
from jax.experimental import pallas as pl
from jax.experimental.pallas import tpu as pltpu
from jax.experimental.pallas import tpu_sc as plsc

B = 32; L = 12; N = 883; C = 3
NP = 896
TOPK = 20
TOD = 288; DOW = 7; SEQ_OUT = 12
HID = 128


_GW = 32
_GCHUNKS = NP // 128


def _sc_gather(table, idx):
    mesh = plsc.VectorSubcoreMesh(core_axis_name="c", subcore_axis_name="s")

    @functools.partial(
        pl.kernel, mesh=mesh,
        out_type=jax.ShapeDtypeStruct((B, NP, 128), jnp.float32),
        scratch_types=[
            pltpu.VMEM((_GCHUNKS, 128), jnp.int32),
            pltpu.VMEM((NP, 128), jnp.float32),
            pltpu.SemaphoreType.DMA,
        ],
    )
    def k(table_hbm, idx_hbm, out_hbm, idx_v, rows_v, sem):
        wid = lax.axis_index("s") * 2 + lax.axis_index("c")
        pltpu.sync_copy(idx_hbm.at[wid], idx_v)
        copies = [
            pltpu.async_copy(table_hbm.at[idx_v.at[j]],
                             rows_v.at[pl.ds(j * 128, 128)], sem)
            for j in range(_GCHUNKS)
        ]
        for c in copies:
            c.wait()
        pltpu.sync_copy(rows_v, out_hbm.at[wid])

    return k(table, idx)


def _kth_largest(x, k, nbits=30):
    rows = x.shape[0]
    kf = jnp.float32(k)

    def body(_, carry):
        lo, hi = carry
        mid = lo + (hi - lo) // 2
        t = lax.bitcast_convert_type(mid, jnp.float32)
        c = jnp.sum((x >= t).astype(jnp.float32), axis=1, keepdims=True)
        ge = c >= kf
        return jnp.where(ge, mid, lo), jnp.where(ge, hi, mid)

    lo0 = jnp.zeros((rows, 1), jnp.int32)
    hi0 = jnp.full((rows, 1), 0x3F800001, jnp.int32)
    lo, _ = lax.fori_loop(0, nbits, body, (lo0, hi0))
    return lax.bitcast_convert_type(lo, jnp.float32)


def _kth_largest_cols(x, k, nbits=30):
    cols = x.shape[1]
    kf = jnp.float32(k)

    def body(_, carry):
        lo, hi = carry
        mid = lo + (hi - lo) // 2
        t = lax.bitcast_convert_type(mid, jnp.float32)
        c = jnp.sum((x >= t).astype(jnp.float32), axis=0, keepdims=True)
        ge = c >= kf
        return jnp.where(ge, mid, lo), jnp.where(ge, hi, mid)

    lo0 = jnp.zeros((1, cols), jnp.int32)
    hi0 = jnp.full((1, cols), 0x3F800001, jnp.int32)
    lo, _ = lax.fori_loop(0, nbits, body, (lo0, hi0))
    return lax.bitcast_convert_type(lo, jnp.float32)


def _main_body(hist_ref, td_ref, ne_ref, neu_ref, ned_ref, e1_ref,
               Wts_ref, bts_ref,
               W1a_ref, b1a_ref, W1b_ref, b1b_ref, W1c_ref, b1c_ref,
               Wf_ref, bf_ref, out_ref, static_scr):
    b = pl.program_id(0)

    @pl.when(b == 0)
    def _():
        r = lax.dot_general(ned_ref[...], neu_ref[...],
                            (((1,), (1,)), ((), ())),
                            preferred_element_type=jnp.float32)
        col = lax.broadcasted_iota(jnp.int32, (NP, NP), 1)
        valid = col < N
        r = jnp.where(valid, jnp.maximum(r, 0.0), -1e30)
        m = jnp.max(r, axis=1, keepdims=True)
        e = jnp.exp(r - m)
        sg = e / jnp.sum(e, axis=1, keepdims=True)
        thr = _kth_largest(sg, TOPK)
        gt = sg > thr
        ties = (sg == thr) & valid
        row_i = lax.broadcasted_iota(jnp.int32, (NP, NP), 0)
        tri = (row_i <= col).astype(jnp.float32)
        rank = lax.dot_general(ties.astype(jnp.float32), tri,
                               (((1,), (0,)), ((), ())),
                               preferred_element_type=jnp.float32)
        need = jnp.float32(TOPK) - jnp.sum(gt.astype(jnp.float32), axis=1,
                                           keepdims=True)
        keep = gt | (ties & (rank <= need))
        eye = (row_i == col).astype(jnp.float32)
        static_scr[...] = jnp.where(keep, sg, 0.0) + eye

    ts = jnp.dot(hist_ref[0], Wts_ref[...],
                 preferred_element_type=jnp.float32) + bts_ref[...]
    H = jnp.concatenate([ts, ne_ref[...], td_ref[0][:, :64]],
                        axis=1)

    h1 = jnp.maximum(jnp.dot(H, W1a_ref[...],
                             preferred_element_type=jnp.float32)
                     + b1a_ref[...], 0.0)
    h2 = jnp.maximum(jnp.dot(h1, W1b_ref[...],
                             preferred_element_type=jnp.float32)
                     + b1b_ref[...], 0.0)
    f1 = jnp.dot(h2, W1c_ref[...],
                 preferred_element_type=jnp.float32) + b1c_ref[...]
    nv = jnp.tanh(e1_ref[...] * f1)

    a = lax.dot_general(nv, nv, (((1,), (1,)), ((), ())),
                        preferred_element_type=jnp.float32)
    adj = jnp.maximum(jnp.tanh(a), 0.0)
    thr = _kth_largest_cols(adj, TOPK)
    dynT = jnp.where(adj >= thr, adj, 0.0)

    hs = jnp.dot(static_scr[...], H, preferred_element_type=jnp.float32)
    hd = lax.dot_general(dynT, H, (((0,), (0,)), ((), ())),
                         preferred_element_type=jnp.float32)
    fused = jnp.maximum(hs + hd, 0.0)
    out_ref[0] = jnp.dot(fused, Wf_ref[...],
                         preferred_element_type=jnp.float32) + bf_ref[...]


def kernel(history_data, TID, DIW, node_emb, node_emb_u, node_emb_d,
           emb1_w, emb2_w, Wts, bts, W1a, b1a, W1b, b1b, W1c, b1c,
           W2a, b2a, W2b, b2b, W2c, b2c, W_fore, b_fore):
    f32 = jnp.float32
    tid_idx = (history_data[:, -1, :, 1] * TOD).astype(jnp.int32)
    diw_idx = (history_data[:, -1, :, 2] * DOW).astype(jnp.int32)
    pad_n = NP - N
    table = jnp.concatenate([
        jnp.broadcast_to(TID[:, None, :], (TOD, DOW, 32)),
        jnp.broadcast_to(DIW[None, :, :], (TOD, DOW, 32)),
    ], axis=-1).reshape(TOD * DOW, 64)
    table = jnp.pad(table, ((0, 0), (0, 64)))
    comb_idx = jnp.pad(tid_idx * DOW + diw_idx,
                       ((0, 0), (0, pad_n))).reshape(B, NP // 128, 128)
    td = _sc_gather(table, comb_idx)
    hist2 = history_data.transpose(0, 2, 1, 3).reshape(B, N, L * C)
    hist2 = jnp.pad(hist2, ((0, 0), (0, pad_n), (0, 0)))
    ne_p = jnp.pad(node_emb, ((0, pad_n), (0, 0)))
    neu_p = jnp.pad(node_emb_u, ((0, pad_n), (0, 0)))
    ned_p = jnp.pad(node_emb_d, ((0, pad_n), (0, 0)))
    e1_p = jnp.pad(emb1_w, ((0, pad_n), (0, 0)))

    full = lambda shape: pl.BlockSpec(shape, lambda b: (0,) * len(shape))
    perb2 = lambda shape: pl.BlockSpec((1,) + shape, lambda b: (b, 0, 0))

    out = pl.pallas_call(
        _main_body,
        grid=(B,),
        in_specs=[
            perb2((NP, L * C)),
            perb2((NP, 128)),
            full((NP, 32)),
            full((NP, 32)),
            full((NP, 32)),
            full((NP, 40)),
            full((L * C, 32)),
            full((1, 32)),
            full((HID, 64)),
            full((1, 64)),
            full((64, 64)),
            full((1, 64)),
            full((64, 40)),
            full((1, 40)),
            full((HID, SEQ_OUT)),
            full((1, SEQ_OUT)),
        ],
        out_specs=perb2((NP, SEQ_OUT)),
        out_shape=jax.ShapeDtypeStruct((B, NP, SEQ_OUT), f32),
        scratch_shapes=[pltpu.VMEM((NP, NP), f32)],
    )(hist2, td, ne_p, neu_p, ned_p, e1_p,
      Wts, bts[None, :], W1a, b1a[None, :], W1b, b1b[None, :],
      W1c, b1c[None, :], W_fore, b_fore[None, :])
    return out[:, :N, :]


# speedup vs baseline: 2.0455x; 1.0111x over previous
import functools

import jax
import jax.numpy as jnp
from jax import lax
from jax.experimental import pallas as pl
from jax.experimental.pallas import tpu as pltpu
from jax.experimental.pallas import tpu_sc as plsc

B = 32; L = 12; N = 883; C = 3
NP = 896
TOPK = 20
TOD = 288; DOW = 7; SEQ_OUT = 12
HID = 128


_GW = 32
_GCHUNKS = NP // 128


def _sc_gather(table, idx):
    mesh = plsc.VectorSubcoreMesh(core_axis_name="c", subcore_axis_name="s")

    @functools.partial(
        pl.kernel, mesh=mesh,
        out_type=jax.ShapeDtypeStruct((B, NP, 128), jnp.float32),
        scratch_types=[
            pltpu.VMEM((_GCHUNKS, 128), jnp.int32),
            pltpu.VMEM((NP, 128), jnp.float32),
            pltpu.SemaphoreType.DMA,
        ],
    )
    def k(table_hbm, idx_hbm, out_hbm, idx_v, rows_v, sem):
        wid = lax.axis_index("s") * 2 + lax.axis_index("c")
        pltpu.sync_copy(idx_hbm.at[wid], idx_v)
        copies = [
            pltpu.async_copy(table_hbm.at[idx_v.at[j]],
                             rows_v.at[pl.ds(j * 128, 128)], sem)
            for j in range(_GCHUNKS)
        ]
        for c in copies:
            c.wait()
        pltpu.sync_copy(rows_v, out_hbm.at[wid])

    return k(table, idx)


def _kth_largest_cols(x, k, nbits=30):
    cols = x.shape[1]
    kf = jnp.float32(k)

    def body(_, carry):
        lo, hi = carry
        mid = lo + (hi - lo) // 2
        t = lax.bitcast_convert_type(mid, jnp.float32)
        c = jnp.sum((x >= t).astype(jnp.float32), axis=0, keepdims=True)
        ge = c >= kf
        return jnp.where(ge, mid, lo), jnp.where(ge, hi, mid)

    lo0 = jnp.zeros((1, cols), jnp.int32)
    hi0 = jnp.full((1, cols), 0x3F800001, jnp.int32)
    lo, _ = lax.fori_loop(0, nbits, body, (lo0, hi0))
    return lax.bitcast_convert_type(lo, jnp.float32)


def _main_body(hist_ref, td_ref, ne_ref, neu_ref, ned_ref, e1_ref,
               Wts_ref, bts_ref,
               W1a_ref, b1a_ref, W1b_ref, b1b_ref, W1c_ref, b1c_ref,
               Wf_ref, bf_ref, out_ref, static_scr):
    b = pl.program_id(0)

    @pl.when(b == 0)
    def _():
        rT = lax.dot_general(neu_ref[...], ned_ref[...],
                             (((1,), (1,)), ((), ())),
                             preferred_element_type=jnp.float32)
        row_i = lax.broadcasted_iota(jnp.int32, (NP, NP), 0)
        col = lax.broadcasted_iota(jnp.int32, (NP, NP), 1)
        valid = row_i < N
        rT = jnp.where(valid, jnp.maximum(rT, 0.0), -1e30)
        m = jnp.max(rT, axis=0, keepdims=True)
        e = jnp.exp(rT - m)
        sgT = e / jnp.sum(e, axis=0, keepdims=True)
        thr = _kth_largest_cols(sgT, TOPK)
        gt = sgT > thr
        ties = (sgT == thr) & valid
        tri = (row_i >= col).astype(jnp.float32)
        rank = lax.dot_general(tri, ties.astype(jnp.float32),
                               (((1,), (0,)), ((), ())),
                               preferred_element_type=jnp.float32)
        need = jnp.float32(TOPK) - jnp.sum(gt.astype(jnp.float32), axis=0,
                                           keepdims=True)
        keep = gt | (ties & (rank <= need))
        eye = (row_i == col).astype(jnp.float32)
        static_scr[...] = jnp.where(keep, sgT, 0.0) + eye

    ts = jnp.dot(hist_ref[0], Wts_ref[...],
                 preferred_element_type=jnp.float32) + bts_ref[...]
    H = jnp.concatenate([ts, ne_ref[...], td_ref[0][:, :64]],
                        axis=1)

    h1 = jnp.maximum(jnp.dot(H, W1a_ref[...],
                             preferred_element_type=jnp.float32)
                     + b1a_ref[...], 0.0)
    h2 = jnp.maximum(jnp.dot(h1, W1b_ref[...],
                             preferred_element_type=jnp.float32)
                     + b1b_ref[...], 0.0)
    f1 = jnp.dot(h2, W1c_ref[...],
                 preferred_element_type=jnp.float32) + b1c_ref[...]
    nv = jnp.tanh(e1_ref[...] * f1)

    a = lax.dot_general(nv, nv, (((1,), (1,)), ((), ())),
                        preferred_element_type=jnp.float32)
    adj = jnp.maximum(jnp.tanh(a), 0.0)
    thr = _kth_largest_cols(adj, TOPK)
    dynT = jnp.where(adj >= thr, adj, 0.0)

    hs = lax.dot_general(static_scr[...], H, (((0,), (0,)), ((), ())),
                         preferred_element_type=jnp.float32)
    hd = lax.dot_general(dynT, H, (((0,), (0,)), ((), ())),
                         preferred_element_type=jnp.float32)
    fused = jnp.maximum(hs + hd, 0.0)
    out_ref[0] = jnp.dot(fused, Wf_ref[...],
                         preferred_element_type=jnp.float32) + bf_ref[...]


def kernel(history_data, TID, DIW, node_emb, node_emb_u, node_emb_d,
           emb1_w, emb2_w, Wts, bts, W1a, b1a, W1b, b1b, W1c, b1c,
           W2a, b2a, W2b, b2b, W2c, b2c, W_fore, b_fore):
    f32 = jnp.float32
    tid_idx = (history_data[:, -1, :, 1] * TOD).astype(jnp.int32)
    diw_idx = (history_data[:, -1, :, 2] * DOW).astype(jnp.int32)
    pad_n = NP - N
    table = jnp.concatenate([
        jnp.broadcast_to(TID[:, None, :], (TOD, DOW, 32)),
        jnp.broadcast_to(DIW[None, :, :], (TOD, DOW, 32)),
    ], axis=-1).reshape(TOD * DOW, 64)
    table = jnp.pad(table, ((0, 0), (0, 64)))
    comb_idx = jnp.pad(tid_idx * DOW + diw_idx,
                       ((0, 0), (0, pad_n))).reshape(B, NP // 128, 128)
    td = _sc_gather(table, comb_idx)
    hist2 = history_data.transpose(0, 2, 1, 3).reshape(B, N, L * C)
    hist2 = jnp.pad(hist2, ((0, 0), (0, pad_n), (0, 0)))
    ne_p = jnp.pad(node_emb, ((0, pad_n), (0, 0)))
    neu_p = jnp.pad(node_emb_u, ((0, pad_n), (0, 0)))
    ned_p = jnp.pad(node_emb_d, ((0, pad_n), (0, 0)))
    e1_p = jnp.pad(emb1_w, ((0, pad_n), (0, 0)))

    full = lambda shape: pl.BlockSpec(shape, lambda b: (0,) * len(shape))
    perb2 = lambda shape: pl.BlockSpec((1,) + shape, lambda b: (b, 0, 0))

    out = pl.pallas_call(
        _main_body,
        grid=(B,),
        in_specs=[
            perb2((NP, L * C)),
            perb2((NP, 128)),
            full((NP, 32)),
            full((NP, 32)),
            full((NP, 32)),
            full((NP, 40)),
            full((L * C, 32)),
            full((1, 32)),
            full((HID, 64)),
            full((1, 64)),
            full((64, 64)),
            full((1, 64)),
            full((64, 40)),
            full((1, 40)),
            full((HID, SEQ_OUT)),
            full((1, SEQ_OUT)),
        ],
        out_specs=perb2((NP, SEQ_OUT)),
        out_shape=jax.ShapeDtypeStruct((B, NP, SEQ_OUT), f32),
        scratch_shapes=[pltpu.VMEM((NP, NP), f32)],
    )(hist2, td, ne_p, neu_p, ned_p, e1_p,
      Wts, bts[None, :], W1a, b1a[None, :], W1b, b1b[None, :],
      W1c, b1c[None, :], W_fore, b_fore[None, :])
    return out[:, :N, :]


# speedup vs baseline: 2.0808x; 1.0173x over previous
import functools

import jax
import jax.numpy as jnp
from jax import lax
from jax.experimental import pallas as pl
from jax.experimental.pallas import tpu as pltpu
from jax.experimental.pallas import tpu_sc as plsc

B = 32; L = 12; N = 883; C = 3
NP = 896
BPG = 2
TOPK = 20
TOD = 288; DOW = 7; SEQ_OUT = 12
HID = 128


_GW = 32
_GCHUNKS = NP // 128


def _sc_gather(table, idx):
    mesh = plsc.VectorSubcoreMesh(core_axis_name="c", subcore_axis_name="s")

    @functools.partial(
        pl.kernel, mesh=mesh,
        out_type=jax.ShapeDtypeStruct((B, NP, 128), jnp.float32),
        scratch_types=[
            pltpu.VMEM((_GCHUNKS, 128), jnp.int32),
            pltpu.VMEM((NP, 128), jnp.float32),
            pltpu.SemaphoreType.DMA,
        ],
    )
    def k(table_hbm, idx_hbm, out_hbm, idx_v, rows_v, sem):
        wid = lax.axis_index("s") * 2 + lax.axis_index("c")
        pltpu.sync_copy(idx_hbm.at[wid], idx_v)
        copies = [
            pltpu.async_copy(table_hbm.at[idx_v.at[j]],
                             rows_v.at[pl.ds(j * 128, 128)], sem)
            for j in range(_GCHUNKS)
        ]
        for c in copies:
            c.wait()
        pltpu.sync_copy(rows_v, out_hbm.at[wid])

    return k(table, idx)


def _kth_largest_cols(x, k, nbits=30):
    cols = x.shape[1]
    kf = jnp.float32(k)

    def body(_, carry):
        lo, hi = carry
        mid = lo + (hi - lo) // 2
        t = lax.bitcast_convert_type(mid, jnp.float32)
        c = jnp.sum((x >= t).astype(jnp.float32), axis=0, keepdims=True)
        ge = c >= kf
        return jnp.where(ge, mid, lo), jnp.where(ge, hi, mid)

    lo0 = jnp.zeros((1, cols), jnp.int32)
    hi0 = jnp.full((1, cols), 0x3F800001, jnp.int32)
    lo, _ = lax.fori_loop(0, nbits, body, (lo0, hi0))
    return lax.bitcast_convert_type(lo, jnp.float32)


def _main_body(hist_ref, td_ref, ne_ref, neu_ref, ned_ref, e1_ref,
               Wts_ref, bts_ref,
               W1a_ref, b1a_ref, W1b_ref, b1b_ref, W1c_ref, b1c_ref,
               Wf_ref, bf_ref, out_ref, static_scr):
    b = pl.program_id(0)

    @pl.when(b == 0)
    def _():
        rT = lax.dot_general(neu_ref[...], ned_ref[...],
                             (((1,), (1,)), ((), ())),
                             preferred_element_type=jnp.float32)
        row_i = lax.broadcasted_iota(jnp.int32, (NP, NP), 0)
        col = lax.broadcasted_iota(jnp.int32, (NP, NP), 1)
        valid = row_i < N
        rT = jnp.where(valid, jnp.maximum(rT, 0.0), -1e30)
        m = jnp.max(rT, axis=0, keepdims=True)
        e = jnp.exp(rT - m)
        sgT = e / jnp.sum(e, axis=0, keepdims=True)
        thr = _kth_largest_cols(sgT, TOPK)
        gt = sgT > thr
        ties = (sgT == thr) & valid
        tri = (row_i >= col).astype(jnp.float32)
        rank = lax.dot_general(tri, ties.astype(jnp.float32),
                               (((1,), (0,)), ((), ())),
                               preferred_element_type=jnp.float32)
        need = jnp.float32(TOPK) - jnp.sum(gt.astype(jnp.float32), axis=0,
                                           keepdims=True)
        keep = gt | (ties & (rank <= need))
        eye = (row_i == col).astype(jnp.float32)
        static_scr[...] = jnp.where(keep, sgT, 0.0) + eye

    for i in range(BPG):
        ts = jnp.dot(hist_ref[i], Wts_ref[...],
                     preferred_element_type=jnp.float32) + bts_ref[...]
        H = jnp.concatenate([ts, ne_ref[...], td_ref[i][:, :64]],
                            axis=1)

        h1 = jnp.maximum(jnp.dot(H, W1a_ref[...],
                                 preferred_element_type=jnp.float32)
                         + b1a_ref[...], 0.0)
        h2 = jnp.maximum(jnp.dot(h1, W1b_ref[...],
                                 preferred_element_type=jnp.float32)
                         + b1b_ref[...], 0.0)
        f1 = jnp.dot(h2, W1c_ref[...],
                     preferred_element_type=jnp.float32) + b1c_ref[...]
        nv = jnp.tanh(e1_ref[...] * f1)

        a = lax.dot_general(nv, nv, (((1,), (1,)), ((), ())),
                            preferred_element_type=jnp.float32)
        adj = jnp.maximum(jnp.tanh(a), 0.0)
        thr = _kth_largest_cols(adj, TOPK)
        dynT = jnp.where(adj >= thr, adj, 0.0)

        hs = lax.dot_general(static_scr[...], H, (((0,), (0,)), ((), ())),
                             preferred_element_type=jnp.float32)
        hd = lax.dot_general(dynT, H, (((0,), (0,)), ((), ())),
                             preferred_element_type=jnp.float32)
        fused = jnp.maximum(hs + hd, 0.0)
        out_ref[i] = jnp.dot(fused, Wf_ref[...],
                             preferred_element_type=jnp.float32) + bf_ref[...]


def kernel(history_data, TID, DIW, node_emb, node_emb_u, node_emb_d,
           emb1_w, emb2_w, Wts, bts, W1a, b1a, W1b, b1b, W1c, b1c,
           W2a, b2a, W2b, b2b, W2c, b2c, W_fore, b_fore):
    f32 = jnp.float32
    tid_idx = (history_data[:, -1, :, 1] * TOD).astype(jnp.int32)
    diw_idx = (history_data[:, -1, :, 2] * DOW).astype(jnp.int32)
    pad_n = NP - N
    table = jnp.concatenate([
        jnp.broadcast_to(TID[:, None, :], (TOD, DOW, 32)),
        jnp.broadcast_to(DIW[None, :, :], (TOD, DOW, 32)),
    ], axis=-1).reshape(TOD * DOW, 64)
    table = jnp.pad(table, ((0, 0), (0, 64)))
    comb_idx = jnp.pad(tid_idx * DOW + diw_idx,
                       ((0, 0), (0, pad_n))).reshape(B, NP // 128, 128)
    td = _sc_gather(table, comb_idx)
    hist2 = history_data.transpose(0, 2, 1, 3).reshape(B, N, L * C)
    hist2 = jnp.pad(hist2, ((0, 0), (0, pad_n), (0, 0)))
    ne_p = jnp.pad(node_emb, ((0, pad_n), (0, 0)))
    neu_p = jnp.pad(node_emb_u, ((0, pad_n), (0, 0)))
    ned_p = jnp.pad(node_emb_d, ((0, pad_n), (0, 0)))
    e1_p = jnp.pad(emb1_w, ((0, pad_n), (0, 0)))

    full = lambda shape: pl.BlockSpec(shape, lambda b: (0,) * len(shape))
    perb2 = lambda shape: pl.BlockSpec((BPG,) + shape, lambda b: (b, 0, 0))

    out = pl.pallas_call(
        _main_body,
        grid=(B // BPG,),
        in_specs=[
            perb2((NP, L * C)),
            perb2((NP, 128)),
            full((NP, 32)),
            full((NP, 32)),
            full((NP, 32)),
            full((NP, 40)),
            full((L * C, 32)),
            full((1, 32)),
            full((HID, 64)),
            full((1, 64)),
            full((64, 64)),
            full((1, 64)),
            full((64, 40)),
            full((1, 40)),
            full((HID, SEQ_OUT)),
            full((1, SEQ_OUT)),
        ],
        out_specs=perb2((NP, SEQ_OUT)),
        out_shape=jax.ShapeDtypeStruct((B, NP, SEQ_OUT), f32),
        scratch_shapes=[pltpu.VMEM((NP, NP), f32)],
    )(hist2, td, ne_p, neu_p, ned_p, e1_p,
      Wts, bts[None, :], W1a, b1a[None, :], W1b, b1b[None, :],
      W1c, b1c[None, :], W_fore, b_fore[None, :])
    return out[:, :N, :]


# speedup vs baseline: 2.1280x; 1.0227x over previous
import functools

import jax
import jax.numpy as jnp
from jax import lax
from jax.experimental import pallas as pl
from jax.experimental.pallas import tpu as pltpu
from jax.experimental.pallas import tpu_sc as plsc

B = 32; L = 12; N = 883; C = 3
NP = 896
BPG = 2
TOPK = 20
TOD = 288; DOW = 7; SEQ_OUT = 12
HID = 128


_GW = 32
_GCHUNKS = NP // 128


def _sc_gather(table, idx):
    mesh = plsc.VectorSubcoreMesh(core_axis_name="c", subcore_axis_name="s")

    @functools.partial(
        pl.kernel, mesh=mesh,
        out_type=jax.ShapeDtypeStruct((B, NP, 128), jnp.float32),
        scratch_types=[
            pltpu.VMEM((_GCHUNKS, 128), jnp.int32),
            pltpu.VMEM((NP, 128), jnp.float32),
            pltpu.SemaphoreType.DMA,
        ],
    )
    def k(table_hbm, idx_hbm, out_hbm, idx_v, rows_v, sem):
        wid = lax.axis_index("s") * 2 + lax.axis_index("c")
        pltpu.sync_copy(idx_hbm.at[wid], idx_v)
        copies = [
            pltpu.async_copy(table_hbm.at[idx_v.at[j]],
                             rows_v.at[pl.ds(j * 128, 128)], sem)
            for j in range(_GCHUNKS)
        ]
        for c in copies:
            c.wait()
        pltpu.sync_copy(rows_v, out_hbm.at[wid])

    return k(table, idx)


def _kth_largest_cols(x, k, nbits=30):
    cols = x.shape[1]
    kf = jnp.float32(k)

    def body(_, carry):
        lo, hi = carry
        mid = lo + (hi - lo) // 2
        t = lax.bitcast_convert_type(mid, jnp.float32)
        c = jnp.sum((x >= t).astype(jnp.float32), axis=0, keepdims=True)
        ge = c >= kf
        return jnp.where(ge, mid, lo), jnp.where(ge, hi, mid)

    lo0 = jnp.zeros((1, cols), jnp.int32)
    hi0 = jnp.full((1, cols), 0x3F800001, jnp.int32)
    lo, _ = lax.fori_loop(0, nbits, body, (lo0, hi0))
    return lax.bitcast_convert_type(lo, jnp.float32)


def _kth_largest_cols_pair(x0, x1, k, nbits=30):
    cols = x0.shape[1]
    kf = jnp.float32(k)

    def body(_, carry):
        lo0, hi0, lo1, hi1 = carry
        mid0 = lo0 + (hi0 - lo0) // 2
        mid1 = lo1 + (hi1 - lo1) // 2
        t0 = lax.bitcast_convert_type(mid0, jnp.float32)
        t1 = lax.bitcast_convert_type(mid1, jnp.float32)
        c0 = jnp.sum((x0 >= t0).astype(jnp.float32), axis=0, keepdims=True)
        c1 = jnp.sum((x1 >= t1).astype(jnp.float32), axis=0, keepdims=True)
        ge0 = c0 >= kf
        ge1 = c1 >= kf
        return (jnp.where(ge0, mid0, lo0), jnp.where(ge0, hi0, mid0),
                jnp.where(ge1, mid1, lo1), jnp.where(ge1, hi1, mid1))

    lo0 = jnp.zeros((1, cols), jnp.int32)
    hi0 = jnp.full((1, cols), 0x3F800001, jnp.int32)
    lo_a, _, lo_b, _ = lax.fori_loop(0, nbits, body, (lo0, hi0, lo0, hi0))
    return (lax.bitcast_convert_type(lo_a, jnp.float32),
            lax.bitcast_convert_type(lo_b, jnp.float32))


def _main_body(hist_ref, td_ref, ne_ref, neu_ref, ned_ref, e1_ref,
               Wts_ref, bts_ref,
               W1a_ref, b1a_ref, W1b_ref, b1b_ref, W1c_ref, b1c_ref,
               Wf_ref, bf_ref, out_ref, static_scr):
    b = pl.program_id(0)

    @pl.when(b == 0)
    def _():
        rT = lax.dot_general(neu_ref[...], ned_ref[...],
                             (((1,), (1,)), ((), ())),
                             preferred_element_type=jnp.float32)
        row_i = lax.broadcasted_iota(jnp.int32, (NP, NP), 0)
        col = lax.broadcasted_iota(jnp.int32, (NP, NP), 1)
        valid = row_i < N
        rT = jnp.where(valid, jnp.maximum(rT, 0.0), -1e30)
        m = jnp.max(rT, axis=0, keepdims=True)
        e = jnp.exp(rT - m)
        sgT = e / jnp.sum(e, axis=0, keepdims=True)
        thr = _kth_largest_cols(sgT, TOPK)
        gt = sgT > thr
        ties = (sgT == thr) & valid
        tri = (row_i >= col).astype(jnp.float32)
        rank = lax.dot_general(tri, ties.astype(jnp.float32),
                               (((1,), (0,)), ((), ())),
                               preferred_element_type=jnp.float32)
        need = jnp.float32(TOPK) - jnp.sum(gt.astype(jnp.float32), axis=0,
                                           keepdims=True)
        keep = gt | (ties & (rank <= need))
        eye = (row_i == col).astype(jnp.float32)
        static_scr[...] = jnp.where(keep, sgT, 0.0) + eye

    Hs, adjs = [], []
    for i in range(BPG):
        ts = jnp.dot(hist_ref[i], Wts_ref[...],
                     preferred_element_type=jnp.float32) + bts_ref[...]
        H = jnp.concatenate([ts, ne_ref[...], td_ref[i][:, :64]],
                            axis=1)

        h1 = jnp.maximum(jnp.dot(H, W1a_ref[...],
                                 preferred_element_type=jnp.float32)
                         + b1a_ref[...], 0.0)
        h2 = jnp.maximum(jnp.dot(h1, W1b_ref[...],
                                 preferred_element_type=jnp.float32)
                         + b1b_ref[...], 0.0)
        f1 = jnp.dot(h2, W1c_ref[...],
                     preferred_element_type=jnp.float32) + b1c_ref[...]
        nv = jnp.tanh(e1_ref[...] * f1)

        a = lax.dot_general(nv, nv, (((1,), (1,)), ((), ())),
                            preferred_element_type=jnp.float32)
        Hs.append(H)
        adjs.append(jnp.maximum(jnp.tanh(a), 0.0))

    thrs = _kth_largest_cols_pair(adjs[0], adjs[1], TOPK)

    for i in range(BPG):
        dynT = jnp.where(adjs[i] >= thrs[i], adjs[i], 0.0)
        hs = lax.dot_general(static_scr[...], Hs[i], (((0,), (0,)), ((), ())),
                             preferred_element_type=jnp.float32)
        hd = lax.dot_general(dynT, Hs[i], (((0,), (0,)), ((), ())),
                             preferred_element_type=jnp.float32)
        fused = jnp.maximum(hs + hd, 0.0)
        out_ref[i] = jnp.dot(fused, Wf_ref[...],
                             preferred_element_type=jnp.float32) + bf_ref[...]


def kernel(history_data, TID, DIW, node_emb, node_emb_u, node_emb_d,
           emb1_w, emb2_w, Wts, bts, W1a, b1a, W1b, b1b, W1c, b1c,
           W2a, b2a, W2b, b2b, W2c, b2c, W_fore, b_fore):
    f32 = jnp.float32
    tid_idx = (history_data[:, -1, :, 1] * TOD).astype(jnp.int32)
    diw_idx = (history_data[:, -1, :, 2] * DOW).astype(jnp.int32)
    pad_n = NP - N
    table = jnp.concatenate([
        jnp.broadcast_to(TID[:, None, :], (TOD, DOW, 32)),
        jnp.broadcast_to(DIW[None, :, :], (TOD, DOW, 32)),
    ], axis=-1).reshape(TOD * DOW, 64)
    table = jnp.pad(table, ((0, 0), (0, 64)))
    comb_idx = jnp.pad(tid_idx * DOW + diw_idx,
                       ((0, 0), (0, pad_n))).reshape(B, NP // 128, 128)
    td = _sc_gather(table, comb_idx)
    hist2 = history_data.transpose(0, 2, 1, 3).reshape(B, N, L * C)
    hist2 = jnp.pad(hist2, ((0, 0), (0, pad_n), (0, 0)))
    ne_p = jnp.pad(node_emb, ((0, pad_n), (0, 0)))
    neu_p = jnp.pad(node_emb_u, ((0, pad_n), (0, 0)))
    ned_p = jnp.pad(node_emb_d, ((0, pad_n), (0, 0)))
    e1_p = jnp.pad(emb1_w, ((0, pad_n), (0, 0)))

    full = lambda shape: pl.BlockSpec(shape, lambda b: (0,) * len(shape))
    perb2 = lambda shape: pl.BlockSpec((BPG,) + shape, lambda b: (b, 0, 0))

    out = pl.pallas_call(
        _main_body,
        grid=(B // BPG,),
        in_specs=[
            perb2((NP, L * C)),
            perb2((NP, 128)),
            full((NP, 32)),
            full((NP, 32)),
            full((NP, 32)),
            full((NP, 40)),
            full((L * C, 32)),
            full((1, 32)),
            full((HID, 64)),
            full((1, 64)),
            full((64, 64)),
            full((1, 64)),
            full((64, 40)),
            full((1, 40)),
            full((HID, SEQ_OUT)),
            full((1, SEQ_OUT)),
        ],
        out_specs=perb2((NP, SEQ_OUT)),
        out_shape=jax.ShapeDtypeStruct((B, NP, SEQ_OUT), f32),
        scratch_shapes=[pltpu.VMEM((NP, NP), f32)],
    )(hist2, td, ne_p, neu_p, ned_p, e1_p,
      Wts, bts[None, :], W1a, b1a[None, :], W1b, b1b[None, :],
      W1c, b1c[None, :], W_fore, b_fore[None, :])
    return out[:, :N, :]
